# Initial kernel scaffold; baseline (speedup 1.0000x reference)
import jax
import jax.numpy as jnp
from jax.experimental import pallas as pl


def kernel(h, edge_index, Wmsg_f, bmsg_f, Wih_f, Whh_f, bih_f, bhh_f, Wmsg_b, bmsg_b, Wih_b, Whh_b, bih_b, bhh_b):
    raise NotImplementedError("write your pallas kernel here")


# speedup vs baseline: 10.3525x; 10.3525x over previous
import functools

import jax
import jax.numpy as jnp
from jax import lax
from jax.experimental import pallas as pl
from jax.experimental.pallas import tpu as pltpu
from jax.experimental.pallas import tpu_sc as plsc

N = 10000
HD = 64
E = 320000
NDIM = 128

NSUB = 16
CH = 100
EPT = E // NSUB
CPT = EPT // CH
ROWS_A = 640
ROWS_B = N - 15 * ROWS_A
D0 = 80


def _sc_scatter(D):
  mesh = plsc.VectorSubcoreMesh(core_axis_name="c", subcore_axis_name="s")

  def body(tf, tb, src_m, dst_m, zrows, gf, gb, idxg, idxs, rows, g_sh, sem):
    c = lax.axis_index("c")
    s = lax.axis_index("s")

    @pl.when(c == 0)
    def _():
      pltpu.sync_copy(src_m.at[pl.ds(s * CPT, CPT)], idxg)
      pltpu.sync_copy(dst_m.at[pl.ds(s * CPT, CPT)], idxs)

    @pl.when(c == 1)
    def _():
      pltpu.sync_copy(dst_m.at[pl.ds(s * CPT, CPT)], idxg)
      pltpu.sync_copy(src_m.at[pl.ds(s * CPT, CPT)], idxs)

    @pl.when(s < 15)
    def _():
      pltpu.sync_copy(zrows, g_sh.at[pl.ds(s * ROWS_A, ROWS_A)])

    @pl.when(s == 15)
    def _():
      pltpu.sync_copy(zrows.at[pl.ds(0, ROWS_B)],
                      g_sh.at[pl.ds(15 * ROWS_A, ROWS_B)])

    plsc.subcore_barrier()

    def run_dir(table, out):
      def chunk(j, carry):
        pltpu.async_copy(table.at[idxg.at[j]], rows, sem).wait()
        pltpu.sync_copy(rows, g_sh.at[idxs.at[j]], add=True)
        return carry

      lax.fori_loop(0, CPT, chunk, 0)
      plsc.subcore_barrier()

      @pl.when(s < 15)
      def _():
        pltpu.sync_copy(g_sh.at[pl.ds(s * ROWS_A, ROWS_A)],
                        out.at[pl.ds(s * ROWS_A, ROWS_A)])

      @pl.when(s == 15)
      def _():
        pltpu.sync_copy(g_sh.at[pl.ds(15 * ROWS_A, ROWS_B)],
                        out.at[pl.ds(15 * ROWS_A, ROWS_B)])

    @pl.when(c == 0)
    def _():
      run_dir(tf, gf)

    @pl.when(c == 1)
    def _():
      run_dir(tb, gb)

  return pl.kernel(
      body,
      out_type=(
          jax.ShapeDtypeStruct((N, D), jnp.float32),
          jax.ShapeDtypeStruct((N, D), jnp.float32),
      ),
      mesh=mesh,
      compiler_params=pltpu.CompilerParams(use_tc_tiling_on_sc=False),
      scratch_types=[
          pltpu.VMEM((CPT, CH), jnp.int32),
          pltpu.VMEM((CPT, CH), jnp.int32),
          pltpu.VMEM((CH, D), jnp.float32),
          pltpu.VMEM_SHARED((N, D), jnp.float32),
          pltpu.SemaphoreType.DMA,
      ],
  )


BN = 2000


def _gru_block(h_ref, g_ref, deg_ref, w1, w2, wih, whh, bm, bih, bhh):
  h = h_ref[...]
  g = g_ref[...][:, :HD]
  deg = deg_ref[...]
  dot = functools.partial(jnp.dot, preferred_element_type=jnp.float32)
  h16 = h.astype(jnp.bfloat16)
  b_part = dot(h16, w2[...]) + bm[...]
  aggr = dot(g, w1[...], precision=lax.Precision.HIGHEST) + deg * b_part
  gi = dot(aggr.astype(jnp.bfloat16), wih[...]) + bih[...]
  gh = dot(h16, whh[...]) + bhh[...]
  r = jax.nn.sigmoid(gi[:, :HD] + gh[:, :HD])
  z = jax.nn.sigmoid(gi[:, HD:2 * HD] + gh[:, HD:2 * HD])
  n = jnp.tanh(gi[:, 2 * HD:] + r * gh[:, 2 * HD:])
  return (1.0 - z) * n + z * h


def _tc_body(final, hf, hb, gf, gb, degf, degb,
             w1f, w2f, wihf, whhf, bmf, bihf, bhhf,
             w1b, w2b, wihb, whhb, bmb, bihb, bhhb,
             *outs):
  hf2 = _gru_block(hf, gf, degf, w1f, w2f, wihf, whhf, bmf, bihf, bhhf)
  hb2 = _gru_block(hb, gb, degb, w1b, w2b, wihb, whhb, bmb, bihb, bhhb)
  if final:
    outs[0][...] = jnp.concatenate([hf2, hb2], axis=1)
  else:
    outs[0][...] = hf2
    outs[1][...] = hb2


def _tc_layer(final, Dg):
  data = lambda cols: pl.BlockSpec((BN, cols), lambda i: (i, 0))
  full = lambda r, c: pl.BlockSpec((r, c), lambda i: (0, 0))
  wspecs = [full(HD, NDIM), full(HD, NDIM), full(NDIM, 3 * HD),
            full(HD, 3 * HD), full(1, NDIM), full(1, 3 * HD),
            full(1, 3 * HD)]
  if final:
    out_specs = [data(NDIM)]
    out_shape = [jax.ShapeDtypeStruct((N, NDIM), jnp.float32)]
  else:
    out_specs = [data(HD)] * 2
    out_shape = [jax.ShapeDtypeStruct((N, HD), jnp.float32)] * 2
  return pl.pallas_call(
      functools.partial(_tc_body, final),
      grid=(N // BN,),
      in_specs=[data(HD)] * 2 + [data(Dg)] * 2 + [data(1)] * 2
               + wspecs + wspecs,
      out_specs=out_specs,
      out_shape=out_shape,
  )


def kernel(h, edge_index, Wmsg_f, bmsg_f, Wih_f, Whh_f, bih_f, bhh_f,
           Wmsg_b, bmsg_b, Wih_b, Whh_b, bih_b, bhh_b):
  hf = h[:, :HD]
  hb = h[:, HD:]
  src_m = edge_index[0].reshape(E // CH, CH)
  dst_m = edge_index[1].reshape(E // CH, CH)

  bf = jnp.bfloat16
  rnd = lambda x: x.astype(bf).astype(jnp.float32)

  def wpack(l, Wm, bm, Wih, Whh, bih, bhh):
    return (rnd(Wm[l][:, :HD].T), Wm[l][:, HD:].T.astype(bf),
            Wih[l].T.astype(bf), Whh[l].T.astype(bf),
            bm[l][None, :], bih[l][None, :], bhh[l][None, :])

  pad = jnp.concatenate(
      [jnp.ones((N, 1), jnp.float32),
       jnp.zeros((N, D0 - HD - 1), jnp.float32)], axis=1)
  tf0 = jnp.concatenate([rnd(hf), pad], axis=1)
  tb0 = jnp.concatenate([rnd(hb), pad], axis=1)

  gf0, gb0 = _sc_scatter(D0)(tf0, tb0, src_m, dst_m,
                             jnp.zeros((ROWS_A, D0), jnp.float32))
  degf = gf0[:, HD:HD + 1]
  degb = gb0[:, HD:HD + 1]

  wf0 = wpack(0, Wmsg_f, bmsg_f, Wih_f, Whh_f, bih_f, bhh_f)
  wb0 = wpack(0, Wmsg_b, bmsg_b, Wih_b, Whh_b, bih_b, bhh_b)
  hf1, hb1 = _tc_layer(False, D0)(hf, hb, gf0, gb0, degf, degb, *wf0, *wb0)

  gf1, gb1 = _sc_scatter(HD)(rnd(hf1), rnd(hb1), src_m, dst_m,
                             jnp.zeros((ROWS_A, HD), jnp.float32))

  wf1 = wpack(1, Wmsg_f, bmsg_f, Wih_f, Whh_f, bih_f, bhh_f)
  wb1 = wpack(1, Wmsg_b, bmsg_b, Wih_b, Whh_b, bih_b, bhh_b)
  (out,) = _tc_layer(True, HD)(hf1, hb1, gf1, gb1, degf, degb, *wf1, *wb1)
  return out


# speedup vs baseline: 20.0962x; 1.9412x over previous
import functools

import jax
import jax.numpy as jnp
from jax import lax
from jax.experimental import pallas as pl
from jax.experimental.pallas import tpu as pltpu
from jax.experimental.pallas import tpu_sc as plsc

N = 10000
HD = 64
E = 320000
NDIM = 128

NSUB = 16
CH = 100
EPT = E // NSUB
CPT = EPT // CH
ROWS_A = 640
ROWS_B = N - 15 * ROWS_A
D0 = 80
NB = 4


def _sc_scatter(D):
  mesh = plsc.VectorSubcoreMesh(core_axis_name="c", subcore_axis_name="s")

  def body(tf, tb, src_m, dst_m, zrows, gf, gb, idxg, idxs, rows, g_sh,
           semg, sems):
    c = lax.axis_index("c")
    s = lax.axis_index("s")

    @pl.when(c == 0)
    def _():
      pltpu.sync_copy(src_m.at[pl.ds(s * CPT, CPT)], idxg)
      pltpu.sync_copy(dst_m.at[pl.ds(s * CPT, CPT)], idxs)

    @pl.when(c == 1)
    def _():
      pltpu.sync_copy(dst_m.at[pl.ds(s * CPT, CPT)], idxg)
      pltpu.sync_copy(src_m.at[pl.ds(s * CPT, CPT)], idxs)

    @pl.when(s < 15)
    def _():
      pltpu.sync_copy(zrows, g_sh.at[pl.ds(s * ROWS_A, ROWS_A)])

    @pl.when(s == 15)
    def _():
      pltpu.sync_copy(zrows.at[pl.ds(0, ROWS_B)],
                      g_sh.at[pl.ds(15 * ROWS_A, ROWS_B)])

    plsc.subcore_barrier()

    def run_dir(table, out):
      for b in range(NB):
        pltpu.async_copy(table.at[idxg.at[b]], rows.at[b], semg)

      def group(i, carry):
        for b in range(NB):
          j = i * NB + b
          pltpu.make_async_copy(table.at[idxg.at[j]], rows.at[b], semg).wait()
          pltpu.async_copy(rows.at[b], g_sh.at[idxs.at[j]], sems, add=True)
          pltpu.make_async_copy(rows.at[b], g_sh.at[idxs.at[j]], sems).wait()

          @pl.when(j < CPT - NB)
          def _():
            pltpu.async_copy(table.at[idxg.at[j + NB]], rows.at[b], semg)

        return carry

      lax.fori_loop(0, CPT // NB, group, 0)
      plsc.subcore_barrier()

      @pl.when(s < 15)
      def _():
        pltpu.sync_copy(g_sh.at[pl.ds(s * ROWS_A, ROWS_A)],
                        out.at[pl.ds(s * ROWS_A, ROWS_A)])

      @pl.when(s == 15)
      def _():
        pltpu.sync_copy(g_sh.at[pl.ds(15 * ROWS_A, ROWS_B)],
                        out.at[pl.ds(15 * ROWS_A, ROWS_B)])

    @pl.when(c == 0)
    def _():
      run_dir(tf, gf)

    @pl.when(c == 1)
    def _():
      run_dir(tb, gb)

  return pl.kernel(
      body,
      out_type=(
          jax.ShapeDtypeStruct((N, D), jnp.float32),
          jax.ShapeDtypeStruct((N, D), jnp.float32),
      ),
      mesh=mesh,
      compiler_params=pltpu.CompilerParams(use_tc_tiling_on_sc=False),
      scratch_types=[
          pltpu.VMEM((CPT, CH), jnp.int32),
          pltpu.VMEM((CPT, CH), jnp.int32),
          pltpu.VMEM((NB, CH, D), jnp.float32),
          pltpu.VMEM_SHARED((N, D), jnp.float32),
          pltpu.SemaphoreType.DMA,
          pltpu.SemaphoreType.DMA,
      ],
  )


BN = 2000


def _gru_block(h_ref, g_ref, deg_ref, w1, w2, wih, whh, bm, bih, bhh):
  h = h_ref[...]
  g = g_ref[...][:, :HD]
  deg = deg_ref[...]
  dot = functools.partial(jnp.dot, preferred_element_type=jnp.float32)
  h16 = h.astype(jnp.bfloat16)
  b_part = dot(h16, w2[...]) + bm[...]
  aggr = dot(g, w1[...], precision=lax.Precision.HIGHEST) + deg * b_part
  gi = dot(aggr.astype(jnp.bfloat16), wih[...]) + bih[...]
  gh = dot(h16, whh[...]) + bhh[...]
  r = jax.nn.sigmoid(gi[:, :HD] + gh[:, :HD])
  z = jax.nn.sigmoid(gi[:, HD:2 * HD] + gh[:, HD:2 * HD])
  n = jnp.tanh(gi[:, 2 * HD:] + r * gh[:, 2 * HD:])
  return (1.0 - z) * n + z * h


def _tc_body(final, hf, hb, gf, gb, degf, degb,
             w1f, w2f, wihf, whhf, bmf, bihf, bhhf,
             w1b, w2b, wihb, whhb, bmb, bihb, bhhb,
             *outs):
  hf2 = _gru_block(hf, gf, degf, w1f, w2f, wihf, whhf, bmf, bihf, bhhf)
  hb2 = _gru_block(hb, gb, degb, w1b, w2b, wihb, whhb, bmb, bihb, bhhb)
  if final:
    outs[0][...] = jnp.concatenate([hf2, hb2], axis=1)
  else:
    outs[0][...] = hf2
    outs[1][...] = hb2
    outs[2][...] = hf2.astype(jnp.bfloat16).astype(jnp.float32)
    outs[3][...] = hb2.astype(jnp.bfloat16).astype(jnp.float32)


def _tc_layer(final, Dg):
  data = lambda cols: pl.BlockSpec((BN, cols), lambda i: (i, 0))
  full = lambda r, c: pl.BlockSpec((r, c), lambda i: (0, 0))
  wspecs = [full(HD, NDIM), full(HD, NDIM), full(NDIM, 3 * HD),
            full(HD, 3 * HD), full(1, NDIM), full(1, 3 * HD),
            full(1, 3 * HD)]
  if final:
    out_specs = [data(NDIM)]
    out_shape = [jax.ShapeDtypeStruct((N, NDIM), jnp.float32)]
  else:
    out_specs = [data(HD)] * 4
    out_shape = [jax.ShapeDtypeStruct((N, HD), jnp.float32)] * 4
  return pl.pallas_call(
      functools.partial(_tc_body, final),
      grid=(N // BN,),
      in_specs=[data(HD)] * 2 + [data(Dg)] * 2 + [data(1)] * 2
               + wspecs + wspecs,
      out_specs=out_specs,
      out_shape=out_shape,
  )


def kernel(h, edge_index, Wmsg_f, bmsg_f, Wih_f, Whh_f, bih_f, bhh_f,
           Wmsg_b, bmsg_b, Wih_b, Whh_b, bih_b, bhh_b):
  hf = h[:, :HD]
  hb = h[:, HD:]
  src_m = edge_index[0].reshape(E // CH, CH)
  dst_m = edge_index[1].reshape(E // CH, CH)

  bf = jnp.bfloat16
  rnd = lambda x: x.astype(bf).astype(jnp.float32)

  def wpack(l, Wm, bm, Wih, Whh, bih, bhh):
    return (rnd(Wm[l][:, :HD].T), Wm[l][:, HD:].T.astype(bf),
            Wih[l].T.astype(bf), Whh[l].T.astype(bf),
            bm[l][None, :], bih[l][None, :], bhh[l][None, :])

  pad = jnp.concatenate(
      [jnp.ones((N, 1), jnp.float32),
       jnp.zeros((N, D0 - HD - 1), jnp.float32)], axis=1)
  tf0 = jnp.concatenate([rnd(hf), pad], axis=1)
  tb0 = jnp.concatenate([rnd(hb), pad], axis=1)

  gf0, gb0 = _sc_scatter(D0)(tf0, tb0, src_m, dst_m,
                             jnp.zeros((ROWS_A, D0), jnp.float32))
  degf = gf0[:, HD:HD + 1]
  degb = gb0[:, HD:HD + 1]

  wf0 = wpack(0, Wmsg_f, bmsg_f, Wih_f, Whh_f, bih_f, bhh_f)
  wb0 = wpack(0, Wmsg_b, bmsg_b, Wih_b, Whh_b, bih_b, bhh_b)
  hf1, hb1, tf1, tb1 = _tc_layer(False, D0)(hf, hb, gf0, gb0, degf, degb,
                                            *wf0, *wb0)

  gf1, gb1 = _sc_scatter(HD)(tf1, tb1, src_m, dst_m,
                             jnp.zeros((ROWS_A, HD), jnp.float32))

  wf1 = wpack(1, Wmsg_f, bmsg_f, Wih_f, Whh_f, bih_f, bhh_f)
  wb1 = wpack(1, Wmsg_b, bmsg_b, Wih_b, Whh_b, bih_b, bhh_b)
  (out,) = _tc_layer(True, HD)(hf1, hb1, gf1, gb1, degf, degb, *wf1, *wb1)
  return out
